# baseline (speedup 1.0000x reference)
import jax
import jax.numpy as jnp
from jax import lax
from jax.experimental import pallas as pl
from jax.experimental.pallas import tpu as pltpu
from jax.experimental.pallas import tpu_sc as plsc

N = 10000
E = 320000
D = 128
DH = 64
NC = 2
NS = 16
CHUNK = 32
NBUF = 4
GRAN = NS * CHUNK * NBUF
E_PAD = ((E + GRAN - 1) // GRAN) * GRAN
EPT = E_PAD // NS
NCHUNK = EPT // CHUNK
N_PAD = 10240
ROWS_PER_TILE = N_PAD // NS
WB_BLK = ROWS_PER_TILE // CHUNK


def _body(xcat, srccat, dst, wb, out, acc, xsp, *bufs):
    c = lax.axis_index("c")
    s = lax.axis_index("s")

    idx_src = bufs[0:NBUF]
    idx_dst = bufs[NBUF:2 * NBUF]
    wv = bufs[2 * NBUF:3 * NBUF]
    rows = bufs[3 * NBUF:4 * NBUF]
    sem_pf = bufs[4 * NBUF:5 * NBUF]
    sem_g = bufs[5 * NBUF:6 * NBUF]

    def zrow(i, _):
        for j in range(DH // 16):
            rows[0][i, pl.ds(j * 16, 16)] = jnp.zeros((16,), jnp.float32)
        return 0
    lax.fori_loop(0, CHUNK, zrow, 0, unroll=4)
    for q in range(WB_BLK):
        sl = pl.ds(s * ROWS_PER_TILE + q * CHUNK, CHUNK)
        pltpu.sync_copy(rows[0], acc.at[sl])
        pltpu.sync_copy(xcat.at[c, sl, :], rows[1])
        pltpu.sync_copy(rows[1], xsp.at[sl])
    plsc.subcore_barrier()

    base_e = s * EPT

    def pf_copies(k, b):
        off = base_e + k * CHUNK
        return (
            (srccat.at[pl.ds(off, CHUNK)], idx_src[b]),
            (dst.at[pl.ds(off, CHUNK)], idx_dst[b]),
            (wb.at[pl.ds(off, CHUNK)], wv[b]),
        )

    def prefetch(k, b):
        for src_ref, dst_ref in pf_copies(k, b):
            pltpu.async_copy(src_ref, dst_ref, sem_pf[b])

    def wait_prefetch(k, b):
        for src_ref, dst_ref in pf_copies(k, b):
            pltpu.make_async_copy(src_ref, dst_ref, sem_pf[b]).wait()

    def issue_gather(b):
        pltpu.async_copy(xsp.at[idx_src[b]], rows[b], sem_g[b])

    def wait_gather(b):
        pltpu.make_async_copy(xsp.at[idx_src[b]], rows[b], sem_g[b]).wait()

    def compute(b):
        r = rows[b]
        w = wv[b]

        def edge(e, _):
            we = w[e, :]
            for j in range(DH // 16):
                sl = pl.ds(j * 16, 16)
                r[e, sl] = r[e, sl] * we
            return 0
        lax.fori_loop(0, CHUNK, edge, 0, unroll=8)

    def step(k, b):
        @pl.when(k + 2 < NCHUNK)
        def _():
            wait_prefetch(k + 2, (b + 2) % NBUF)
            issue_gather((b + 2) % NBUF)
        wait_gather(b)
        compute(b)
        pltpu.sync_copy(rows[b], acc.at[idx_dst[b]], add=True)

        @pl.when(k + NBUF < NCHUNK)
        def _():
            prefetch(k + NBUF, b)

    for b in range(NBUF):
        prefetch(b, b)
    for b in range(2):
        wait_prefetch(b, b)
        issue_gather(b)

    def group(p, _):
        for b in range(NBUF):
            step(NBUF * p + b, b)
        return 0
    lax.fori_loop(0, NCHUNK // NBUF, group, 0)

    plsc.subcore_barrier()
    for q in range(WB_BLK):
        sl = pl.ds(s * ROWS_PER_TILE + q * CHUNK, CHUNK)
        b = q % 2
        pltpu.sync_copy(acc.at[sl], rows[b])
        pltpu.sync_copy(rows[b], out.at[c, sl, :])


@jax.jit
def _spmm(xcat, srccat, dst, wb):
    mesh = plsc.VectorSubcoreMesh(
        core_axis_name="c", subcore_axis_name="s", num_cores=NC, num_subcores=NS
    )
    scratch = [
        pltpu.VMEM_SHARED((N_PAD, DH), jnp.float32),
        pltpu.VMEM_SHARED((N_PAD, DH), jnp.float32),
    ]
    scratch += [pltpu.VMEM((CHUNK,), jnp.int32) for _ in range(NBUF)]
    scratch += [pltpu.VMEM((CHUNK,), jnp.int32) for _ in range(NBUF)]
    scratch += [pltpu.VMEM((CHUNK, 16), jnp.float32) for _ in range(NBUF)]
    scratch += [pltpu.VMEM((CHUNK, DH), jnp.float32) for _ in range(NBUF)]
    scratch += [pltpu.SemaphoreType.DMA for _ in range(2 * NBUF)]
    f = pl.kernel(
        _body,
        out_type=jax.ShapeDtypeStruct((NC, N_PAD, DH), jnp.float32),
        mesh=mesh,
        scratch_types=scratch,
        compiler_params=pltpu.CompilerParams(use_tc_tiling_on_sc=False),
    )
    return f(xcat, srccat, dst, wb)


def kernel(x, edge_index, edge_weight):
    dst = edge_index[0]
    src = edge_index[1]
    pad = E_PAD - E
    xcat = jnp.stack([x[:, :DH], x[:, DH:]])
    xcat = jnp.pad(xcat, ((0, 0), (0, N_PAD - N), (0, 0)))
    srccat = jnp.pad(src, (0, pad))
    dst_p = jnp.pad(dst, (0, pad))
    w_p = jnp.pad(edge_weight, (0, pad))
    wb = jnp.broadcast_to(w_p[:, None], (E_PAD, 16))
    o = _spmm(xcat, srccat, dst_p, wb)
    return jnp.concatenate([o[0, :N], o[1, :N]], axis=1)


# speedup vs baseline: 1.1170x; 1.1170x over previous
import jax
import jax.numpy as jnp
from jax import lax
from jax.experimental import pallas as pl
from jax.experimental.pallas import tpu as pltpu
from jax.experimental.pallas import tpu_sc as plsc

N = 10000
E = 320000
D = 128
DH = 64
NC = 2
NS = 16
CHUNK = 64
NBUF = 4
GRAN = NS * CHUNK * NBUF
E_PAD = ((E + GRAN - 1) // GRAN) * GRAN
EPT = E_PAD // NS
NCHUNK = EPT // CHUNK
N_PAD = 10240
ROWS_PER_TILE = N_PAD // NS
WB_BLK = ROWS_PER_TILE // CHUNK


def _body(xcat, srccat, dst, wb, out, acc, xsp, *bufs):
    c = lax.axis_index("c")
    s = lax.axis_index("s")

    idx_src = bufs[0:NBUF]
    idx_dst = bufs[NBUF:2 * NBUF]
    wv = bufs[2 * NBUF:3 * NBUF]
    rows = bufs[3 * NBUF:4 * NBUF]
    sem_pf = bufs[4 * NBUF:5 * NBUF]
    sem_g = bufs[5 * NBUF:6 * NBUF]
    sem_sc = bufs[6 * NBUF:7 * NBUF]

    def zrow(i, _):
        for j in range(DH // 16):
            rows[0][i, pl.ds(j * 16, 16)] = jnp.zeros((16,), jnp.float32)
        return 0
    lax.fori_loop(0, CHUNK, zrow, 0, unroll=4)
    for q in range(WB_BLK):
        sl = pl.ds(s * ROWS_PER_TILE + q * CHUNK, CHUNK)
        pltpu.sync_copy(rows[0], acc.at[sl])
        pltpu.sync_copy(xcat.at[c, sl, :], rows[1])
        pltpu.sync_copy(rows[1], xsp.at[sl])
    plsc.subcore_barrier()

    base_e = s * EPT

    def pf_copies(k, b):
        off = base_e + k * CHUNK
        return (
            (srccat.at[pl.ds(off, CHUNK)], idx_src[b]),
            (dst.at[pl.ds(off, CHUNK)], idx_dst[b]),
            (wb.at[pl.ds(off, CHUNK)], wv[b]),
        )

    def prefetch(k, b):
        for src_ref, dst_ref in pf_copies(k, b):
            pltpu.async_copy(src_ref, dst_ref, sem_pf[b])

    def wait_prefetch(k, b):
        for src_ref, dst_ref in pf_copies(k, b):
            pltpu.make_async_copy(src_ref, dst_ref, sem_pf[b]).wait()

    def issue_gather(b):
        pltpu.async_copy(xsp.at[idx_src[b]], rows[b], sem_g[b])

    def wait_gather(b):
        pltpu.make_async_copy(xsp.at[idx_src[b]], rows[b], sem_g[b]).wait()

    def compute(b):
        r = rows[b]
        w = wv[b]

        def edge(e, _):
            we = w[e, :]
            for j in range(DH // 16):
                sl = pl.ds(j * 16, 16)
                r[e, sl] = r[e, sl] * we
            return 0
        lax.fori_loop(0, CHUNK, edge, 0, unroll=8)

    def issue_scatter(b):
        pltpu.async_copy(rows[b], acc.at[idx_dst[b]], sem_sc[b], add=True)

    def wait_scatter(b):
        pltpu.make_async_copy(rows[b], acc.at[idx_dst[b]], sem_sc[b]).wait()

    def step(k, b):
        bp = (b - 1) % NBUF

        @pl.when(k + 2 < NCHUNK)
        def _():
            wait_prefetch(k + 2, (b + 2) % NBUF)
            issue_gather((b + 2) % NBUF)
        wait_gather(b)
        compute(b)

        @pl.when(k > 0)
        def _():
            wait_scatter(bp)

            @pl.when(k - 1 + NBUF < NCHUNK)
            def _():
                prefetch(k - 1 + NBUF, bp)
        issue_scatter(b)

    for b in range(NBUF):
        prefetch(b, b)
    for b in range(2):
        wait_prefetch(b, b)
        issue_gather(b)

    def group(p, _):
        for b in range(NBUF):
            step(NBUF * p + b, b)
        return 0
    lax.fori_loop(0, NCHUNK // NBUF, group, 0)
    wait_scatter((NCHUNK - 1) % NBUF)

    plsc.subcore_barrier()
    for q in range(WB_BLK):
        sl = pl.ds(s * ROWS_PER_TILE + q * CHUNK, CHUNK)
        b = q % 2
        pltpu.sync_copy(acc.at[sl], rows[b])
        pltpu.sync_copy(rows[b], out.at[c, sl, :])


@jax.jit
def _spmm(xcat, srccat, dst, wb):
    mesh = plsc.VectorSubcoreMesh(
        core_axis_name="c", subcore_axis_name="s", num_cores=NC, num_subcores=NS
    )
    scratch = [
        pltpu.VMEM_SHARED((N_PAD, DH), jnp.float32),
        pltpu.VMEM_SHARED((N_PAD, DH), jnp.float32),
    ]
    scratch += [pltpu.VMEM((CHUNK,), jnp.int32) for _ in range(NBUF)]
    scratch += [pltpu.VMEM((CHUNK,), jnp.int32) for _ in range(NBUF)]
    scratch += [pltpu.VMEM((CHUNK, 16), jnp.float32) for _ in range(NBUF)]
    scratch += [pltpu.VMEM((CHUNK, DH), jnp.float32) for _ in range(NBUF)]
    scratch += [pltpu.SemaphoreType.DMA for _ in range(3 * NBUF)]
    f = pl.kernel(
        _body,
        out_type=jax.ShapeDtypeStruct((NC, N_PAD, DH), jnp.float32),
        mesh=mesh,
        scratch_types=scratch,
        compiler_params=pltpu.CompilerParams(use_tc_tiling_on_sc=False),
    )
    return f(xcat, srccat, dst, wb)


def kernel(x, edge_index, edge_weight):
    dst = edge_index[0]
    src = edge_index[1]
    pad = E_PAD - E
    xcat = jnp.stack([x[:, :DH], x[:, DH:]])
    xcat = jnp.pad(xcat, ((0, 0), (0, N_PAD - N), (0, 0)))
    srccat = jnp.pad(src, (0, pad))
    dst_p = jnp.pad(dst, (0, pad))
    w_p = jnp.pad(edge_weight, (0, pad))
    wb = jnp.broadcast_to(w_p[:, None], (E_PAD, 16))
    o = _spmm(xcat, srccat, dst_p, wb)
    return jnp.concatenate([o[0, :N], o[1, :N]], axis=1)


# speedup vs baseline: 1.3570x; 1.2148x over previous
import jax
import jax.numpy as jnp
from jax import lax
from jax.experimental import pallas as pl
from jax.experimental.pallas import tpu as pltpu
from jax.experimental.pallas import tpu_sc as plsc

N = 10000
E = 320000
D = 128
DH = 64
NC = 2
NS = 16
CHUNK = 64
NBUF = 4
GRAN = NS * CHUNK * NBUF
E_PAD = ((E + GRAN - 1) // GRAN) * GRAN
EPT = E_PAD // NS
NCHUNK = EPT // CHUNK
N_PAD = 10240
ROWS_PER_TILE = N_PAD // NS
WB_BLK = ROWS_PER_TILE // CHUNK


def _body(xcat, srccat, dst, wb, out, acc, xsp, *bufs):
    c = lax.axis_index("c")
    s = lax.axis_index("s")

    idx_src = bufs[0:NBUF]
    idx_dst = bufs[NBUF:2 * NBUF]
    wv = bufs[2 * NBUF:3 * NBUF]
    rows = bufs[3 * NBUF:4 * NBUF]
    sem_pf = bufs[4 * NBUF:5 * NBUF]
    sem_g = bufs[5 * NBUF:6 * NBUF]
    sem_sc = bufs[6 * NBUF:7 * NBUF]

    def zrow(i, _):
        for j in range(DH // 16):
            rows[0][i, pl.ds(j * 16, 16)] = jnp.zeros((16,), jnp.float32)
        return 0
    lax.fori_loop(0, CHUNK, zrow, 0, unroll=4)
    for q in range(WB_BLK):
        sl = pl.ds(s * ROWS_PER_TILE + q * CHUNK, CHUNK)
        pltpu.sync_copy(rows[0], acc.at[sl])
        pltpu.sync_copy(xcat.at[c, sl, :], rows[1])
        pltpu.sync_copy(rows[1], xsp.at[sl])
    plsc.subcore_barrier()

    base_e = s * EPT

    def pf_copies(k, b):
        off = base_e + k * CHUNK
        return (
            (srccat.at[pl.ds(off, CHUNK)], idx_src[b]),
            (dst.at[pl.ds(off, CHUNK)], idx_dst[b]),
            (wb.at[pl.ds(off, CHUNK)], wv[b]),
        )

    def prefetch(k, b):
        for src_ref, dst_ref in pf_copies(k, b):
            pltpu.async_copy(src_ref, dst_ref, sem_pf[b])

    def wait_prefetch(k, b):
        for src_ref, dst_ref in pf_copies(k, b):
            pltpu.make_async_copy(src_ref, dst_ref, sem_pf[b]).wait()

    def issue_gather(b):
        pltpu.async_copy(xsp.at[idx_src[b]], rows[b], sem_g[b])

    def wait_gather(b):
        pltpu.make_async_copy(xsp.at[idx_src[b]], rows[b], sem_g[b]).wait()

    def compute(b):
        r = rows[b]
        w = wv[b]

        def edge(e, _):
            we = w[e, :]
            for j in range(DH // 16):
                sl = pl.ds(j * 16, 16)
                r[e, sl] = r[e, sl] * we
            return 0
        lax.fori_loop(0, CHUNK, edge, 0, unroll=8)

    def issue_scatter(b):
        pltpu.async_copy(rows[b], acc.at[idx_dst[b]], sem_sc[b], add=True)

    def wait_scatter(b):
        pltpu.make_async_copy(rows[b], acc.at[idx_dst[b]], sem_sc[b]).wait()

    def step(k, b):
        bp = (b - 1) % NBUF

        @pl.when(k + 2 < NCHUNK)
        def _():
            wait_prefetch(k + 2, (b + 2) % NBUF)
            issue_gather((b + 2) % NBUF)
        del bp
        wait_gather(b)
        compute(b)
        pltpu.sync_copy(rows[b], acc.at[idx_dst[b]], add=True)

        @pl.when(k + NBUF < NCHUNK)
        def _():
            prefetch(k + NBUF, b)

    for b in range(NBUF):
        prefetch(b, b)
    for b in range(2):
        wait_prefetch(b, b)
        issue_gather(b)

    def group(p, _):
        for b in range(NBUF):
            step(NBUF * p + b, b)
        return 0
    lax.fori_loop(0, NCHUNK // NBUF, group, 0)

    plsc.subcore_barrier()
    for q in range(WB_BLK):
        sl = pl.ds(s * ROWS_PER_TILE + q * CHUNK, CHUNK)
        b = q % 2
        pltpu.sync_copy(acc.at[sl], rows[b])
        pltpu.sync_copy(rows[b], out.at[c, sl, :])


@jax.jit
def _spmm(xcat, srccat, dst, wb):
    mesh = plsc.VectorSubcoreMesh(
        core_axis_name="c", subcore_axis_name="s", num_cores=NC, num_subcores=NS
    )
    scratch = [
        pltpu.VMEM_SHARED((N_PAD, DH), jnp.float32),
        pltpu.VMEM_SHARED((N_PAD, DH), jnp.float32),
    ]
    scratch += [pltpu.VMEM((CHUNK,), jnp.int32) for _ in range(NBUF)]
    scratch += [pltpu.VMEM((CHUNK,), jnp.int32) for _ in range(NBUF)]
    scratch += [pltpu.VMEM((CHUNK, 16), jnp.float32) for _ in range(NBUF)]
    scratch += [pltpu.VMEM((CHUNK, DH), jnp.float32) for _ in range(NBUF)]
    scratch += [pltpu.SemaphoreType.DMA for _ in range(3 * NBUF)]
    f = pl.kernel(
        _body,
        out_type=jax.ShapeDtypeStruct((NC, N_PAD, DH), jnp.float32),
        mesh=mesh,
        scratch_types=scratch,
        compiler_params=pltpu.CompilerParams(use_tc_tiling_on_sc=False),
    )
    return f(xcat, srccat, dst, wb)


def kernel(x, edge_index, edge_weight):
    dst = edge_index[0]
    src = edge_index[1]
    pad = E_PAD - E
    xcat = jnp.stack([x[:, :DH], x[:, DH:]])
    xcat = jnp.pad(xcat, ((0, 0), (0, N_PAD - N), (0, 0)))
    srccat = jnp.pad(src, (0, pad))
    dst_p = jnp.pad(dst, (0, pad))
    w_p = jnp.pad(edge_weight, (0, pad))
    wb = jnp.broadcast_to(w_p[:, None], (E_PAD, 16))
    o = _spmm(xcat, srccat, dst_p, wb)
    return jnp.concatenate([o[0, :N], o[1, :N]], axis=1)
